# baseline (speedup 1.0000x reference)
import functools
import math

import jax
import jax.numpy as jnp
from jax import lax
from jax.experimental import pallas as pl
from jax.experimental.pallas import tpu as pltpu
from jax.experimental.pallas import tpu_sc as plsc

T_TOK = 16384
N_GRAPH = 16
D = 4
EMB = 64
NS = 16
CHUNK = T_TOK // NS
GROUPS = CHUNK // 16


def _tc_body(xt_ref, t_ref, fw_ref, linw_ref, linb_ref, vnet_ref,
             b2_ref, pq_ref, ctab_ref):
    a = vnet_ref[pl.ds(0, D), :]
    w2 = vnet_ref[pl.ds(D, EMB), :]
    x0t = xt_ref[pl.ds(0, D), :]
    x1t = xt_ref[pl.ds(D, D), :]
    dxt = x0t - x1t
    dn = (((0,), (0,)), ((), ()))
    pt = lax.dot_general(a, dxt, dn, preferred_element_type=jnp.float32)
    qt = lax.dot_general(a, x1t, dn, preferred_element_type=jnp.float32) - dxt
    pq_ref[pl.ds(0, D), :] = pt
    pq_ref[pl.ds(D, D), :] = qt

    t = t_ref[...]
    proj = t * (fw_ref[...][None, :] * (2.0 * math.pi))
    emb = jnp.concatenate([jnp.sin(proj), jnp.cos(proj)], axis=1)
    temb = jnp.dot(emb, linw_ref[...], preferred_element_type=jnp.float32)
    temb = temb + linb_ref[...][None, :]
    c = jnp.dot(temb, w2, preferred_element_type=jnp.float32)
    c = c + b2_ref[...][None, :]
    ctab_ref[...] = jnp.concatenate(
        [t, c, jnp.zeros((N_GRAPH, 3), jnp.float32)], axis=1)


_tc_call = pl.pallas_call(
    _tc_body,
    out_shape=[jax.ShapeDtypeStruct((2 * D, T_TOK), jnp.float32),
               jax.ShapeDtypeStruct((N_GRAPH, 8), jnp.float32)],
)


def _sc_body(pq_hbm, seg_hbm, ctab_hbm,
             dist_hbm, pw_hbm,
             pqv, segv, ctv, accv, shared, allv, odv, opv, sem):
    sid = lax.axis_index("s")
    base = sid * CHUNK
    cps = [pltpu.async_copy(pq_hbm.at[:, pl.ds(base, CHUNK)], pqv, sem),
           pltpu.async_copy(seg_hbm.at[pl.ds(base, CHUNK)], segv, sem),
           pltpu.async_copy(ctab_hbm, ctv, sem)]
    for cp in cps:
        cp.wait()

    iota = lax.iota(jnp.int32, 16)
    col0 = iota >> 4

    def group(j):
        seg = segv[pl.ds(j * 16, 16)]
        tv = plsc.load_gather(ctv, [seg, col0])
        out = []
        for d in range(D):
            cd = plsc.load_gather(ctv, [seg, col0 + (1 + d)])
            p = pqv[d, pl.ds(j * 16, 16)]
            q = pqv[D + d, pl.ds(j * 16, 16)]
            diff = tv * p + (q + cd)
            out.append(diff * diff)
        return tuple(out)

    @plsc.parallel_loop(1, GROUPS, carry=group(0), unroll=4)
    def acc(j, carry):
        g = group(j)
        return tuple(carry[d] + g[d] for d in range(D))

    for d in range(D):
        accv[pl.ds(d * 16, 16)] = acc[d]
    pltpu.sync_copy(accv, shared.at[pl.ds(sid * 64, 64)])
    plsc.subcore_barrier()

    @pl.when(sid == 0)
    def _():
        pltpu.sync_copy(shared, allv)

        def rbody(r, carry):
            return tuple(carry[d] + allv[pl.ds(r * 64 + d * 16, 16)]
                         for d in range(D))

        tot = lax.fori_loop(1, NS, rbody,
                            tuple(allv[pl.ds(d * 16, 16)] for d in range(D)))
        scale = 1.0 / (2.0 * T_TOK)
        s = [jnp.sum(tot[d]) * scale for d in range(D)]
        m = [((iota + 1) == (d + 1)).astype(jnp.float32) for d in range(D)]
        pw = m[0] * s[0] + m[1] * s[1] + m[2] * s[2] + m[3] * s[3]
        dist = (s[0] + s[1] + s[2] + s[3]) * 0.25
        opv[...] = pw
        odv[...] = jnp.broadcast_to(dist, (16,))
        pltpu.sync_copy(opv, pw_hbm)
        pltpu.sync_copy(odv, dist_hbm)


_sc_call = functools.partial(
    pl.kernel,
    out_type=[jax.ShapeDtypeStruct((16,), jnp.float32),
              jax.ShapeDtypeStruct((16,), jnp.float32)],
    mesh=plsc.VectorSubcoreMesh(core_axis_name="c", subcore_axis_name="s",
                                num_cores=1),
    compiler_params=pltpu.CompilerParams(needs_layout_passes=False),
    scratch_types=[
        pltpu.VMEM((2 * D, CHUNK), jnp.float32),
        pltpu.VMEM((CHUNK,), jnp.int32),
        pltpu.VMEM((N_GRAPH, 8), jnp.float32),
        pltpu.VMEM((64,), jnp.float32),
        pltpu.VMEM_SHARED((NS * 64,), jnp.float32),
        pltpu.VMEM((NS * 64,), jnp.float32),
        pltpu.VMEM((16,), jnp.float32),
        pltpu.VMEM((16,), jnp.float32),
        pltpu.SemaphoreType.DMA,
    ],
)(_sc_body)


def kernel(x0, x1, t_graph, seg_ids, fourier_W, lin_W, lin_b, vnet_W, vnet_b):
    xt = jnp.concatenate([x0.T, x1.T], axis=0)
    pq, ctab = _tc_call(xt, t_graph, fourier_W, lin_W, lin_b, vnet_W, vnet_b)
    dist, pw = _sc_call(pq, seg_ids, ctab)
    return dist[0], pw[:D]


# speedup vs baseline: 1.0058x; 1.0058x over previous
import functools
import math

import jax
import jax.numpy as jnp
from jax import lax
from jax.experimental import pallas as pl
from jax.experimental.pallas import tpu as pltpu
from jax.experimental.pallas import tpu_sc as plsc

T_TOK = 16384
N_GRAPH = 16
D = 4
EMB = 64
NS = 16
CHUNK = T_TOK // NS
GROUPS = CHUNK // 16


def _tc_body(xt_ref, t_ref, fw_ref, linw_ref, linb_ref, vnet_ref,
             b2_ref, pq_ref, ctab_ref):
    a = vnet_ref[pl.ds(0, D), :]
    w2 = vnet_ref[pl.ds(D, EMB), :]
    x0t = xt_ref[pl.ds(0, D), :]
    x1t = xt_ref[pl.ds(D, D), :]
    dxt = x0t - x1t
    dn = (((0,), (0,)), ((), ()))
    pt = lax.dot_general(a, dxt, dn, preferred_element_type=jnp.float32)
    qt = lax.dot_general(a, x1t, dn, preferred_element_type=jnp.float32) - dxt
    pq_ref[pl.ds(0, D), :] = pt
    pq_ref[pl.ds(D, D), :] = qt

    t = t_ref[...]
    proj = t * (fw_ref[...][None, :] * (2.0 * math.pi))
    emb = jnp.concatenate([jnp.sin(proj), jnp.cos(proj)], axis=1)
    temb = jnp.dot(emb, linw_ref[...], preferred_element_type=jnp.float32)
    temb = temb + linb_ref[...][None, :]
    c = jnp.dot(temb, w2, preferred_element_type=jnp.float32)
    c = c + b2_ref[...][None, :]
    ctab_ref[...] = jnp.concatenate(
        [t, c, jnp.zeros((N_GRAPH, 3), jnp.float32)], axis=1)


_tc_call = pl.pallas_call(
    _tc_body,
    out_shape=[jax.ShapeDtypeStruct((2 * D, T_TOK), jnp.float32),
               jax.ShapeDtypeStruct((N_GRAPH, 8), jnp.float32)],
)


def _sc_body(pq_hbm, seg_hbm, ctab_hbm,
             dist_hbm, pw_hbm,
             pqv, segv, ctv, accv, shared, allv, odv, opv, sem):
    sid = lax.axis_index("s")
    base = sid * CHUNK
    cps = [pltpu.async_copy(pq_hbm.at[:, pl.ds(base, CHUNK)], pqv, sem),
           pltpu.async_copy(seg_hbm.at[pl.ds(base, CHUNK)], segv, sem),
           pltpu.async_copy(ctab_hbm, ctv, sem)]
    for cp in cps:
        cp.wait()

    iota = lax.iota(jnp.int32, 16)
    col0 = iota >> 4

    def group(j):
        seg = segv[pl.ds(j * 16, 16)]
        tv = plsc.load_gather(ctv, [seg, col0])
        out = []
        for d in range(D):
            cd = plsc.load_gather(ctv, [seg, col0 + (1 + d)])
            p = pqv[d, pl.ds(j * 16, 16)]
            q = pqv[D + d, pl.ds(j * 16, 16)]
            diff = tv * p + (q + cd)
            out.append(diff * diff)
        return tuple(out)

    def body(j, carry):
        g = group(j)
        return tuple(carry[d] + g[d] for d in range(D))

    acc = lax.fori_loop(1, GROUPS, body, group(0))

    for d in range(D):
        accv[pl.ds(d * 16, 16)] = acc[d]
    pltpu.sync_copy(accv, shared.at[pl.ds(sid * 64, 64)])
    plsc.subcore_barrier()

    @pl.when(sid == 0)
    def _():
        pltpu.sync_copy(shared, allv)

        def rbody(r, carry):
            return tuple(carry[d] + allv[pl.ds(r * 64 + d * 16, 16)]
                         for d in range(D))

        tot = lax.fori_loop(1, NS, rbody,
                            tuple(allv[pl.ds(d * 16, 16)] for d in range(D)))
        scale = 1.0 / (2.0 * T_TOK)
        s = [jnp.sum(tot[d]) * scale for d in range(D)]
        m = [((iota + 1) == (d + 1)).astype(jnp.float32) for d in range(D)]
        pw = m[0] * s[0] + m[1] * s[1] + m[2] * s[2] + m[3] * s[3]
        dist = (s[0] + s[1] + s[2] + s[3]) * 0.25
        opv[...] = pw
        odv[...] = jnp.broadcast_to(dist, (16,))
        pltpu.sync_copy(opv, pw_hbm)
        pltpu.sync_copy(odv, dist_hbm)


_sc_call = functools.partial(
    pl.kernel,
    out_type=[jax.ShapeDtypeStruct((16,), jnp.float32),
              jax.ShapeDtypeStruct((16,), jnp.float32)],
    mesh=plsc.VectorSubcoreMesh(core_axis_name="c", subcore_axis_name="s",
                                num_cores=1),
    compiler_params=pltpu.CompilerParams(needs_layout_passes=False),
    scratch_types=[
        pltpu.VMEM((2 * D, CHUNK), jnp.float32),
        pltpu.VMEM((CHUNK,), jnp.int32),
        pltpu.VMEM((N_GRAPH, 8), jnp.float32),
        pltpu.VMEM((64,), jnp.float32),
        pltpu.VMEM_SHARED((NS * 64,), jnp.float32),
        pltpu.VMEM((NS * 64,), jnp.float32),
        pltpu.VMEM((16,), jnp.float32),
        pltpu.VMEM((16,), jnp.float32),
        pltpu.SemaphoreType.DMA,
    ],
)(_sc_body)


def kernel(x0, x1, t_graph, seg_ids, fourier_W, lin_W, lin_b, vnet_W, vnet_b):
    xt = jnp.concatenate([x0.T, x1.T], axis=0)
    pq, ctab = _tc_call(xt, t_graph, fourier_W, lin_W, lin_b, vnet_W, vnet_b)
    dist, pw = _sc_call(pq, seg_ids, ctab)
    return dist[0], pw[:D]
